# Initial kernel scaffold; baseline (speedup 1.0000x reference)
import jax
import jax.numpy as jnp
from jax.experimental import pallas as pl


def kernel(x_src, x_dst, edge_label_index):
    raise NotImplementedError("write your pallas kernel here")


# speedup vs baseline: 3.3750x; 3.3750x over previous
import functools

import jax
import jax.numpy as jnp
from jax import lax
from jax.experimental import pallas as pl
from jax.experimental.pallas import tpu as pltpu
from jax.experimental.pallas import tpu_sc as plsc

N_NODES = 10000
N_EDGES = 320000
D_FEAT = 128

NC = 2
NS = 16
NW = NC * NS
EW = N_EDGES // NW

C = 400
NCHUNK = EW // C
SUB = 80
NSUB = C // SUB


def _dot_kernel(src_idx, dst_idx, xsrc, xdst, out, siv, div, ra, rb, ov,
                sem_a, sem_b):
    wid = lax.axis_index("s") * NC + lax.axis_index("c")
    base = wid * EW

    def chunk_body(i, carry):
        cb = base + i * C
        pltpu.sync_copy(src_idx.at[pl.ds(cb, C)], siv)
        pltpu.sync_copy(dst_idx.at[pl.ds(cb, C)], div)
        copies = []
        for j in range(NSUB):
            copies.append(
                pltpu.async_copy(xsrc.at[siv.at[pl.ds(j * SUB, SUB)]],
                                 ra.at[pl.ds(j * SUB, SUB)], sem_a))
            copies.append(
                pltpu.async_copy(xdst.at[div.at[pl.ds(j * SUB, SUB)]],
                                 rb.at[pl.ds(j * SUB, SUB)], sem_b))
        for cp in copies:
            cp.wait()

        lane = lax.iota(jnp.int32, 16)

        def group_body(g, gcarry):
            res = jnp.zeros((16,), jnp.float32)
            for u in range(16):
                e = g * 16 + u
                acc0 = ra[e, pl.ds(0, 16)] * rb[e, pl.ds(0, 16)]
                acc1 = ra[e, pl.ds(16, 16)] * rb[e, pl.ds(16, 16)]
                acc2 = ra[e, pl.ds(32, 16)] * rb[e, pl.ds(32, 16)]
                acc3 = ra[e, pl.ds(48, 16)] * rb[e, pl.ds(48, 16)]
                acc0 = acc0 + ra[e, pl.ds(64, 16)] * rb[e, pl.ds(64, 16)]
                acc1 = acc1 + ra[e, pl.ds(80, 16)] * rb[e, pl.ds(80, 16)]
                acc2 = acc2 + ra[e, pl.ds(96, 16)] * rb[e, pl.ds(96, 16)]
                acc3 = acc3 + ra[e, pl.ds(112, 16)] * rb[e, pl.ds(112, 16)]
                acc = (acc0 + acc1) + (acc2 + acc3)
                res = jnp.where(lane == u, jnp.sum(acc), res)
            ov[pl.ds(g * 16, 16)] = res
            return gcarry

        lax.fori_loop(0, C // 16, group_body, 0)
        pltpu.sync_copy(ov, out.at[pl.ds(cb, C)])
        return carry

    lax.fori_loop(0, NCHUNK, chunk_body, 0)


_mesh = plsc.VectorSubcoreMesh(core_axis_name="c", subcore_axis_name="s")

_kernel_call = functools.partial(
    pl.kernel,
    mesh=_mesh,
    compiler_params=pltpu.CompilerParams(needs_layout_passes=False),
    out_type=jax.ShapeDtypeStruct((N_EDGES,), jnp.float32),
    scratch_types=[
        pltpu.VMEM((C,), jnp.int32),
        pltpu.VMEM((C,), jnp.int32),
        pltpu.VMEM((C, D_FEAT), jnp.float32),
        pltpu.VMEM((C, D_FEAT), jnp.float32),
        pltpu.VMEM((C,), jnp.float32),
        pltpu.SemaphoreType.DMA,
        pltpu.SemaphoreType.DMA,
    ],
)(_dot_kernel)


@jax.jit
def kernel(x_src, x_dst, edge_label_index):
    eli = edge_label_index.astype(jnp.int32)
    return _kernel_call(eli[0], eli[1], x_src, x_dst)
